# baseline (speedup 1.0000x reference)
import jax
import jax.numpy as jnp
from jax import lax
from jax.experimental import pallas as pl
from jax.experimental.pallas import tpu as pltpu
from jax.experimental.pallas import tpu_sc as plsc

_B = 16
_L = 2048
_D = 768
_LRAW = _L + 2
_CH = 64
_ZCH = 32
_NCH = _L // _CH
_NBUF = 2
_MASKP = 2064

_mesh = plsc.VectorSubcoreMesh(
    core_axis_name="c", subcore_axis_name="s", num_cores=2, num_subcores=16)


def _sc_body(emb, maskp, posp, zsrc,
             ctx_hbm, asp_hbm, clen_hbm, alen_hbm,
             buf0, buf1, zbuf, idx0, idx1, idxp, maskv, posv, lenv,
             sg0, sg1, ss0, ss1, sz, sp, spz):
    cidx = lax.axis_index("c")
    sidx = lax.axis_index("s")
    wid = sidx * 2 + cidx
    b = wid & 15
    is_ctx = wid < 16
    bufs = [buf0, buf1]
    idxs = [idx0, idx1]
    sg = [sg0, sg1]
    ss = [ss0, ss1]
    lanes = jnp.arange(16, dtype=jnp.int32)

    pltpu.sync_copy(zsrc, zbuf)
    pltpu.sync_copy(posp.at[b], posv)
    pv = posv[...]
    left = pv[0]
    right = pv[1]

    pltpu.sync_copy(maskp.at[b], maskv)

    def _mstep(i, acc):
        chunk = maskv[pl.ds(i * 16, 16)]
        return acc + jnp.where(chunk != 0, 1, 0).astype(jnp.int32)

    acc = lax.fori_loop(0, _MASKP // 16, _mstep, jnp.zeros((16,), jnp.int32))
    s_count = jnp.sum(acc)

    nv = jnp.where(is_ctx, jnp.clip(s_count - 2, 0, _L),
                   jnp.clip(right - left, 0, _L))
    src0 = jnp.where(is_ctx, 1, left)
    lenval = jnp.where(is_ctx, s_count, right - left)

    lenv[...] = jnp.broadcast_to(lenval, (16,))

    @pl.when(is_ctx)
    def _():
        pltpu.sync_copy(lenv, clen_hbm.at[b])

    @pl.when(jnp.logical_not(is_ctx))
    def _():
        pltpu.sync_copy(lenv, alen_hbm.at[b])

    def write_idx(iref, base, clamp_hi):
        for q in range(_CH // 16):
            v = jnp.minimum(base + q * 16 + lanes, clamp_hi)
            iref[pl.ds(q * 16, 16)] = v

    def do_region(out_hbm):
        nfull = nv // _CH
        m = nv - nfull * _CH
        have_m = m > 0
        zc0 = nfull + jnp.where(have_m, 1, 0)

        @pl.when(have_m)
        def _():
            base = nfull * _CH
            for h in range(_CH // _ZCH):
                pltpu.make_async_copy(
                    zbuf,
                    out_hbm.at[b, pl.ds(pl.multiple_of(base + h * _ZCH, _ZCH),
                                        _ZCH)], spz).start()

        def zfill(i, carry):
            off = pl.multiple_of(zc0 * _CH + i * _ZCH, _ZCH)
            pltpu.make_async_copy(
                zbuf, out_hbm.at[b, pl.ds(off, _ZCH)], sz).start()
            return carry

        nz = (_NCH - zc0) * (_CH // _ZCH)
        lax.fori_loop(0, nz, zfill, 0)

        def g_start(i, j):
            write_idx(idxs[j], src0 + i * _CH, _LRAW - 1)
            pltpu.make_async_copy(
                emb.at[b].at[idxs[j]], bufs[j], sg[j]).start()

        def g_wait(j):
            pltpu.make_async_copy(
                emb.at[b].at[idxs[j]], bufs[j], sg[j]).wait()

        def s_start(i, j):
            pltpu.make_async_copy(
                bufs[j],
                out_hbm.at[b, pl.ds(pl.multiple_of(i * _CH, _CH), _CH)],
                ss[j]).start()

        def s_wait(j):
            pltpu.make_async_copy(
                bufs[j], out_hbm.at[b, pl.ds(0, _CH)], ss[j]).wait()

        for j in range(_NBUF):
            @pl.when(j < nfull)
            def _(j=j):
                g_start(j, j)

        def ring(it, carry):
            g = it * _NBUF
            for j in range(_NBUF):
                i = g + j

                @pl.when(i < nfull)
                def _(i=i, j=j):
                    g_wait(j)
                    s_start(i, j)

                @pl.when(i + _NBUF < nfull)
                def _(i=i, j=j):
                    s_wait(j)
                    g_start(i + _NBUF, j)
            return carry

        lax.fori_loop(0, (nfull + _NBUF - 1) // _NBUF, ring, 0)
        for j in range(_NBUF):
            @pl.when(j < nfull)
            def _(j=j):
                s_wait(j)

        @pl.when(have_m)
        def _():
            write_idx(idxp, src0 + nfull * _CH, src0 + nv - 1)
            pltpu.make_async_copy(emb.at[b].at[idxp], buf0, sp).start()
            pltpu.make_async_copy(emb.at[b].at[idxp], buf0, sp).wait()
            for h in range(_CH // _ZCH):
                pltpu.make_async_copy(
                    zbuf, out_hbm.at[b, pl.ds(0, _ZCH)], spz).wait()
            write_idx(idxp, nfull * _CH, nv - 1)
            pltpu.make_async_copy(buf0, out_hbm.at[b].at[idxp], sp).start()
            pltpu.make_async_copy(buf0, out_hbm.at[b].at[idxp], sp).wait()

        def zwait(i, carry):
            pltpu.make_async_copy(
                zbuf, out_hbm.at[b, pl.ds(0, _ZCH)], sz).wait()
            return carry

        lax.fori_loop(0, nz, zwait, 0)

    @pl.when(is_ctx)
    def _():
        do_region(ctx_hbm)

    @pl.when(jnp.logical_not(is_ctx))
    def _():
        do_region(asp_hbm)


@jax.jit
def kernel(ctx_embeddings, text_mask, aspect_positions):
    maskp = jnp.pad(text_mask, ((0, 0), (0, _MASKP - _LRAW)))
    posp = jnp.pad(aspect_positions, ((0, 0), (0, 14)))
    zsrc = jnp.zeros((_ZCH, _D), jnp.float32)

    sc_call = pl.kernel(
        _sc_body,
        out_type=[
            jax.ShapeDtypeStruct((_B, _L, _D), jnp.float32),
            jax.ShapeDtypeStruct((_B, _L, _D), jnp.float32),
            jax.ShapeDtypeStruct((_B, 16), jnp.int32),
            jax.ShapeDtypeStruct((_B, 16), jnp.int32),
        ],
        mesh=_mesh,
        compiler_params=pltpu.CompilerParams(needs_layout_passes=False),
        scratch_types=[
            pltpu.VMEM((_CH, _D), jnp.float32),
            pltpu.VMEM((_CH, _D), jnp.float32),
            pltpu.VMEM((_ZCH, _D), jnp.float32),
            pltpu.VMEM((_CH,), jnp.int32),
            pltpu.VMEM((_CH,), jnp.int32),
            pltpu.VMEM((_CH,), jnp.int32),
            pltpu.VMEM((_MASKP,), jnp.int32),
            pltpu.VMEM((16,), jnp.int32),
            pltpu.VMEM((16,), jnp.int32),
        ] + [pltpu.SemaphoreType.DMA] * 7,
    )
    ctx, asp, clen, alen = sc_call(ctx_embeddings, maskp, posp, zsrc)
    return (ctx, asp, clen[:, 0], alen[:, 0])


# speedup vs baseline: 1.0562x; 1.0562x over previous
import jax
import jax.numpy as jnp
from jax import lax
from jax.experimental import pallas as pl
from jax.experimental.pallas import tpu as pltpu
from jax.experimental.pallas import tpu_sc as plsc

_B = 16
_L = 2048
_D = 768
_LRAW = _L + 2
_CH = 32
_ZCH = 32
_NCH = _L // _CH
_NBUF = 3
_MASKP = 2064

_mesh = plsc.VectorSubcoreMesh(
    core_axis_name="c", subcore_axis_name="s", num_cores=2, num_subcores=16)


def _sc_body(emb, maskp, posp, zsrc,
             ctx_hbm, asp_hbm, clen_hbm, alen_hbm,
             buf0, buf1, buf2, zbuf, idx0, idx1, idx2, idxp, maskv, posv,
             lenv, sg0, sg1, sg2, ss0, ss1, ss2, sz, sp, spz):
    cidx = lax.axis_index("c")
    sidx = lax.axis_index("s")
    wid = sidx * 2 + cidx
    b = wid & 15
    is_ctx = wid < 16
    bufs = [buf0, buf1, buf2]
    idxs = [idx0, idx1, idx2]
    sg = [sg0, sg1, sg2]
    ss = [ss0, ss1, ss2]
    lanes = jnp.arange(16, dtype=jnp.int32)

    pltpu.sync_copy(zsrc, zbuf)
    pltpu.sync_copy(posp.at[b], posv)
    pv = posv[...]
    left = pv[0]
    right = pv[1]

    pltpu.sync_copy(maskp.at[b], maskv)

    def _mstep(i, acc):
        chunk = maskv[pl.ds(i * 16, 16)]
        return acc + jnp.where(chunk != 0, 1, 0).astype(jnp.int32)

    acc = lax.fori_loop(0, _MASKP // 16, _mstep, jnp.zeros((16,), jnp.int32))
    s_count = jnp.sum(acc)

    nv = jnp.where(is_ctx, jnp.clip(s_count - 2, 0, _L),
                   jnp.clip(right - left, 0, _L))
    src0 = jnp.where(is_ctx, 1, left)
    lenval = jnp.where(is_ctx, s_count, right - left)

    lenv[...] = jnp.broadcast_to(lenval, (16,))

    @pl.when(is_ctx)
    def _():
        pltpu.sync_copy(lenv, clen_hbm.at[b])

    @pl.when(jnp.logical_not(is_ctx))
    def _():
        pltpu.sync_copy(lenv, alen_hbm.at[b])

    def write_idx(iref, base, clamp_hi):
        for q in range(_CH // 16):
            v = jnp.minimum(base + q * 16 + lanes, clamp_hi)
            iref[pl.ds(q * 16, 16)] = v

    def do_region(out_hbm):
        nfull = nv // _CH
        m = nv - nfull * _CH
        have_m = m > 0
        zc0 = nfull + jnp.where(have_m, 1, 0)

        @pl.when(have_m)
        def _():
            base = nfull * _CH
            for h in range(_CH // _ZCH):
                pltpu.make_async_copy(
                    zbuf,
                    out_hbm.at[b, pl.ds(pl.multiple_of(base + h * _ZCH, _ZCH),
                                        _ZCH)], spz).start()

        def zfill(i, carry):
            off = pl.multiple_of(zc0 * _CH + i * _ZCH, _ZCH)
            pltpu.make_async_copy(
                zbuf, out_hbm.at[b, pl.ds(off, _ZCH)], sz).start()
            return carry

        nz = (_NCH - zc0) * (_CH // _ZCH)
        lax.fori_loop(0, nz, zfill, 0)

        def g_start(i, j):
            write_idx(idxs[j], src0 + i * _CH, _LRAW - 1)
            pltpu.make_async_copy(
                emb.at[b].at[idxs[j]], bufs[j], sg[j]).start()

        def g_wait(j):
            pltpu.make_async_copy(
                emb.at[b].at[idxs[j]], bufs[j], sg[j]).wait()

        def s_start(i, j):
            pltpu.make_async_copy(
                bufs[j],
                out_hbm.at[b, pl.ds(pl.multiple_of(i * _CH, _CH), _CH)],
                ss[j]).start()

        def s_wait(j):
            pltpu.make_async_copy(
                bufs[j], out_hbm.at[b, pl.ds(0, _CH)], ss[j]).wait()

        for j in range(_NBUF):
            @pl.when(j < nfull)
            def _(j=j):
                g_start(j, j)

        def ring(it, carry):
            g = it * _NBUF
            for j in range(_NBUF):
                i = g + j

                @pl.when(i < nfull)
                def _(i=i, j=j):
                    g_wait(j)
                    s_start(i, j)

                @pl.when(i + _NBUF < nfull)
                def _(i=i, j=j):
                    s_wait(j)
                    g_start(i + _NBUF, j)
            return carry

        lax.fori_loop(0, (nfull + _NBUF - 1) // _NBUF, ring, 0)
        for j in range(_NBUF):
            @pl.when(j < nfull)
            def _(j=j):
                s_wait(j)

        @pl.when(have_m)
        def _():
            write_idx(idxp, src0 + nfull * _CH, src0 + nv - 1)
            pltpu.make_async_copy(emb.at[b].at[idxp], buf0, sp).start()
            pltpu.make_async_copy(emb.at[b].at[idxp], buf0, sp).wait()
            for h in range(_CH // _ZCH):
                pltpu.make_async_copy(
                    zbuf, out_hbm.at[b, pl.ds(0, _ZCH)], spz).wait()
            write_idx(idxp, nfull * _CH, nv - 1)
            pltpu.make_async_copy(buf0, out_hbm.at[b].at[idxp], sp).start()
            pltpu.make_async_copy(buf0, out_hbm.at[b].at[idxp], sp).wait()

        def zwait(i, carry):
            pltpu.make_async_copy(
                zbuf, out_hbm.at[b, pl.ds(0, _ZCH)], sz).wait()
            return carry

        lax.fori_loop(0, nz, zwait, 0)

    @pl.when(is_ctx)
    def _():
        do_region(ctx_hbm)

    @pl.when(jnp.logical_not(is_ctx))
    def _():
        do_region(asp_hbm)


@jax.jit
def kernel(ctx_embeddings, text_mask, aspect_positions):
    maskp = jnp.pad(text_mask, ((0, 0), (0, _MASKP - _LRAW)))
    posp = jnp.pad(aspect_positions, ((0, 0), (0, 14)))
    zsrc = jnp.zeros((_ZCH, _D), jnp.float32)

    sc_call = pl.kernel(
        _sc_body,
        out_type=[
            jax.ShapeDtypeStruct((_B, _L, _D), jnp.float32),
            jax.ShapeDtypeStruct((_B, _L, _D), jnp.float32),
            jax.ShapeDtypeStruct((_B, 16), jnp.int32),
            jax.ShapeDtypeStruct((_B, 16), jnp.int32),
        ],
        mesh=_mesh,
        compiler_params=pltpu.CompilerParams(needs_layout_passes=False),
        scratch_types=[
            pltpu.VMEM((_CH, _D), jnp.float32),
            pltpu.VMEM((_CH, _D), jnp.float32),
            pltpu.VMEM((_CH, _D), jnp.float32),
            pltpu.VMEM((_ZCH, _D), jnp.float32),
            pltpu.VMEM((_CH,), jnp.int32),
            pltpu.VMEM((_CH,), jnp.int32),
            pltpu.VMEM((_CH,), jnp.int32),
            pltpu.VMEM((_CH,), jnp.int32),
            pltpu.VMEM((_MASKP,), jnp.int32),
            pltpu.VMEM((16,), jnp.int32),
            pltpu.VMEM((16,), jnp.int32),
        ] + [pltpu.SemaphoreType.DMA] * 9,
    )
    ctx, asp, clen, alen = sc_call(ctx_embeddings, maskp, posp, zsrc)
    return (ctx, asp, clen[:, 0], alen[:, 0])
